# initial kernel scaffold (unmeasured)
import jax
import jax.numpy as jnp
from jax import lax
from jax.experimental import pallas as pl
from jax.experimental.pallas import tpu as pltpu


def kernel(Q, K, V):
    b, s, h, d = K.shape

    def body(k_ref, v_ref, kout_ref, vout_ref, local_sems, send_sems, recv_sems):
        my_x = lax.axis_index("x")
        my_y = lax.axis_index("y")
        other_x = 1 - my_x

        copy_k = pltpu.make_async_copy(k_ref, kout_ref.at[my_x], local_sems.at[0])
        copy_v = pltpu.make_async_copy(v_ref, vout_ref.at[my_x], local_sems.at[1])
        copy_k.start()
        copy_v.start()

        rdma_k = pltpu.make_async_remote_copy(
            src_ref=k_ref,
            dst_ref=kout_ref.at[my_x],
            send_sem=send_sems.at[0],
            recv_sem=recv_sems.at[0],
            device_id=(other_x, my_y),
            device_id_type=pl.DeviceIdType.MESH,
        )
        rdma_v = pltpu.make_async_remote_copy(
            src_ref=v_ref,
            dst_ref=vout_ref.at[my_x],
            send_sem=send_sems.at[1],
            recv_sem=recv_sems.at[1],
            device_id=(other_x, my_y),
            device_id_type=pl.DeviceIdType.MESH,
        )
        rdma_k.start()
        rdma_v.start()

        copy_k.wait()
        copy_v.wait()
        rdma_k.wait()
        rdma_v.wait()

    kout, vout = pl.pallas_call(
        body,
        out_shape=[
            jax.ShapeDtypeStruct((2, b, s, h, d), K.dtype),
            jax.ShapeDtypeStruct((2, b, s, h, d), V.dtype),
        ],
        in_specs=[
            pl.BlockSpec(memory_space=pltpu.VMEM),
            pl.BlockSpec(memory_space=pltpu.VMEM),
        ],
        out_specs=[
            pl.BlockSpec(memory_space=pltpu.VMEM),
            pl.BlockSpec(memory_space=pltpu.VMEM),
        ],
        scratch_shapes=[
            pltpu.SemaphoreType.DMA((2,)),
            pltpu.SemaphoreType.DMA((2,)),
            pltpu.SemaphoreType.DMA((2,)),
        ],
        compiler_params=pltpu.CompilerParams(collective_id=0),
    )(K, V)

    K_full = jnp.concatenate([kout[0], kout[1]], axis=1)
    V_full = jnp.concatenate([vout[0], vout[1]], axis=1)

    scale = d ** -0.5
    S = jnp.einsum("bqhd,bkhd->bhqk", Q, K_full) * scale
    m = jnp.max(S, axis=-1, keepdims=True)
    P = jnp.exp(S - m)
    P = P / jnp.sum(P, axis=-1, keepdims=True)
    return jnp.einsum("bhqk,bkhd->bqhd", P, V_full).astype(jnp.float32)


# baseline (device time: 270434 ns/iter reference)
import jax
import jax.numpy as jnp
from jax import lax
from jax.experimental import pallas as pl
from jax.experimental.pallas import tpu as pltpu


def kernel(Q, K, V):
    b, s, h, d = K.shape

    def body(k_ref, v_ref, kout_ref, vout_ref, local_sems, send_sems, recv_sems):
        my_x = lax.axis_index("x")
        my_y = lax.axis_index("y")
        other_x = 1 - my_x

        copy_k = pltpu.make_async_copy(k_ref, kout_ref.at[my_x], local_sems.at[0])
        copy_v = pltpu.make_async_copy(v_ref, vout_ref.at[my_x], local_sems.at[1])
        copy_k.start()
        copy_v.start()

        rdma_k = pltpu.make_async_remote_copy(
            src_ref=k_ref,
            dst_ref=kout_ref.at[my_x],
            send_sem=send_sems.at[0],
            recv_sem=recv_sems.at[0],
            device_id=(other_x, my_y),
            device_id_type=pl.DeviceIdType.MESH,
        )
        rdma_v = pltpu.make_async_remote_copy(
            src_ref=v_ref,
            dst_ref=vout_ref.at[my_x],
            send_sem=send_sems.at[1],
            recv_sem=recv_sems.at[1],
            device_id=(other_x, my_y),
            device_id_type=pl.DeviceIdType.MESH,
        )
        rdma_k.start()
        rdma_v.start()

        copy_k.wait()
        copy_v.wait()
        rdma_k.wait()
        rdma_v.wait()

    kout, vout = pl.pallas_call(
        body,
        out_shape=[
            jax.ShapeDtypeStruct((2, b, s, h, d), K.dtype),
            jax.ShapeDtypeStruct((2, b, s, h, d), V.dtype),
        ],
        in_specs=[
            pl.BlockSpec(memory_space=pltpu.VMEM),
            pl.BlockSpec(memory_space=pltpu.VMEM),
        ],
        out_specs=[
            pl.BlockSpec(memory_space=pltpu.VMEM),
            pl.BlockSpec(memory_space=pltpu.VMEM),
        ],
        scratch_shapes=[
            pltpu.SemaphoreType.DMA((2,)),
            pltpu.SemaphoreType.DMA((2,)),
            pltpu.SemaphoreType.DMA((2,)),
        ],
    )(K, V)

    K_full = jnp.concatenate([kout[0], kout[1]], axis=1)
    V_full = jnp.concatenate([vout[0], vout[1]], axis=1)

    scale = d ** -0.5
    S = jnp.einsum("bqhd,bkhd->bhqk", Q, K_full) * scale
    m = jnp.max(S, axis=-1, keepdims=True)
    P = jnp.exp(S - m)
    P = P / jnp.sum(P, axis=-1, keepdims=True)
    return jnp.einsum("bhqk,bkhd->bqhd", P, V_full).astype(jnp.float32)


# device time: 179690 ns/iter; 1.5050x vs baseline; 1.5050x over previous
import jax
import jax.numpy as jnp
from jax import lax
from jax.experimental import pallas as pl
from jax.experimental.pallas import tpu as pltpu

C = 8


def kernel(Q, K, V):
    b, s, h, d = K.shape
    half = s // 2
    cs = half // C

    def body(k_ref, v_ref, kout_ref, vout_ref,
             local_sems, sx_sems, rx_sems, sy_sems, ry_sems):
        my_x = lax.axis_index("x")
        my_y = lax.axis_index("y")
        other = 1 - my_x
        nx = (other, my_y)
        ny = (my_x, 1 - my_y)

        barrier = pltpu.get_barrier_semaphore()
        for nbr in (nx, ny):
            pl.semaphore_signal(barrier, inc=1, device_id=nbr,
                                device_id_type=pl.DeviceIdType.MESH)
        pl.semaphore_wait(barrier, 2)

        cpk = pltpu.make_async_copy(
            k_ref, kout_ref.at[:, pl.ds(my_x * s, s)], local_sems.at[0])
        cpv = pltpu.make_async_copy(
            v_ref, vout_ref.at[:, pl.ds(my_x * s, s)], local_sems.at[1])
        cpk.start()
        cpv.start()

        my_off = my_y * half
        oth_off = (1 - my_y) * half
        tensors = ((k_ref, kout_ref), (v_ref, vout_ref))

        x_sends = []
        for c in range(C):
            off = my_off + c * cs
            for t, (src, dst) in enumerate(tensors):
                r = pltpu.make_async_remote_copy(
                    src_ref=src.at[:, pl.ds(off, cs)],
                    dst_ref=dst.at[:, pl.ds(my_x * s + off, cs)],
                    send_sem=sx_sems.at[t, c],
                    recv_sem=rx_sems.at[t, c],
                    device_id=nx,
                    device_id_type=pl.DeviceIdType.MESH,
                )
                r.start()
                x_sends.append(r)

        y_fwds = []
        for c in range(C):
            off = my_off + c * cs
            for t, (src, dst) in enumerate(tensors):
                recv = pltpu.make_async_remote_copy(
                    src_ref=src.at[:, pl.ds(off, cs)],
                    dst_ref=dst.at[:, pl.ds(other * s + off, cs)],
                    send_sem=sx_sems.at[t, c],
                    recv_sem=rx_sems.at[t, c],
                    device_id=nx,
                    device_id_type=pl.DeviceIdType.MESH,
                )
                recv.wait_recv()
                f = pltpu.make_async_remote_copy(
                    src_ref=dst.at[:, pl.ds(other * s + off, cs)],
                    dst_ref=dst.at[:, pl.ds(other * s + off, cs)],
                    send_sem=sy_sems.at[t, c],
                    recv_sem=ry_sems.at[t, c],
                    device_id=ny,
                    device_id_type=pl.DeviceIdType.MESH,
                )
                f.start()
                y_fwds.append(f)

        for c in range(C):
            off = oth_off + c * cs
            for t, (src, dst) in enumerate(tensors):
                rv = pltpu.make_async_remote_copy(
                    src_ref=src.at[:, pl.ds(off, cs)],
                    dst_ref=dst.at[:, pl.ds(other * s + off, cs)],
                    send_sem=sy_sems.at[t, c],
                    recv_sem=ry_sems.at[t, c],
                    device_id=ny,
                    device_id_type=pl.DeviceIdType.MESH,
                )
                rv.wait_recv()

        for r in x_sends:
            r.wait_send()
        for f in y_fwds:
            f.wait_send()
        cpk.wait()
        cpv.wait()

    kout, vout = pl.pallas_call(
        body,
        out_shape=[
            jax.ShapeDtypeStruct((b, 2 * s, h, d), K.dtype),
            jax.ShapeDtypeStruct((b, 2 * s, h, d), V.dtype),
        ],
        in_specs=[
            pl.BlockSpec(memory_space=pltpu.VMEM),
            pl.BlockSpec(memory_space=pltpu.VMEM),
        ],
        out_specs=[
            pl.BlockSpec(memory_space=pltpu.VMEM),
            pl.BlockSpec(memory_space=pltpu.VMEM),
        ],
        scratch_shapes=[
            pltpu.SemaphoreType.DMA((2,)),
            pltpu.SemaphoreType.DMA((2, C)),
            pltpu.SemaphoreType.DMA((2, C)),
            pltpu.SemaphoreType.DMA((2, C)),
            pltpu.SemaphoreType.DMA((2, C)),
        ],
        compiler_params=pltpu.CompilerParams(collective_id=0),
    )(K, V)

    scale = d ** -0.5
    S = jnp.einsum("bqhd,bkhd->bhqk", Q, kout) * scale
    m = jnp.max(S, axis=-1, keepdims=True)
    P = jnp.exp(S - m)
    P = P / jnp.sum(P, axis=-1, keepdims=True)
    return jnp.einsum("bhqk,bkhd->bqhd", P, vout).astype(jnp.float32)
